# baseline (speedup 1.0000x reference)
import functools

import jax
import jax.numpy as jnp
from jax import lax
from jax.experimental import pallas as pl
from jax.experimental.pallas import tpu as pltpu
from jax.experimental.pallas import tpu_sc as plsc


def _make_scatter(E, N, D, NC, NS, K, NB, RK=80):
    W = NC * NS
    EPW = E // W
    C = EPW // K
    T = EPW - C * K
    NG = (C + NB - 1) // NB
    RCH = N // RK
    mesh = plsc.VectorSubcoreMesh(core_axis_name="c", subcore_axis_name="s")

    @functools.partial(
        pl.kernel,
        out_type=jax.ShapeDtypeStruct((NC, N, D), jnp.float32),
        mesh=mesh,
        scratch_types=[
            pltpu.VMEM((NB, K), jnp.int32),
            pltpu.VMEM((NB, K, D), jnp.float32),
            pltpu.VMEM((16,), jnp.int32),
            pltpu.VMEM((RK, D), jnp.float32),
            pltpu.VMEM_SHARED((N, D), jnp.float32),
            pltpu.SemaphoreType.DMA((NB,)),
            pltpu.SemaphoreType.DMA((NB,)),
        ],
    )
    def scatter_kernel(col_hbm, ea_hbm, out_hbm, idx_v, rows_v, idx_t, zbuf,
                       acc, load_sem, sc_sem):
        c = lax.axis_index("c")
        s = lax.axis_index("s")
        wid = c * NS + s
        rbase = wid * EPW
        cbase = E + rbase

        def issue_loads(j, b):
            c0 = pl.multiple_of(cbase + j * K, 8)
            r0 = pl.multiple_of(rbase + j * K, 8)
            pltpu.async_copy(col_hbm.at[pl.ds(c0, K)], idx_v.at[b],
                             load_sem.at[b])
            pltpu.async_copy(ea_hbm.at[pl.ds(r0, K)], rows_v.at[b],
                             load_sem.at[b])

        def wait_loads(b):
            pltpu.make_async_copy(col_hbm.at[pl.ds(0, K)], idx_v.at[b],
                                  load_sem.at[b]).wait()
            pltpu.make_async_copy(ea_hbm.at[pl.ds(0, K)], rows_v.at[b],
                                  load_sem.at[b]).wait()

        def wait_scatter(b):
            pltpu.make_async_copy(rows_v.at[b], acc.at[idx_v.at[b]],
                                  sc_sem.at[b]).wait()

        for b in range(NB):
            issue_loads(b, b)

        z = jnp.zeros((16,), jnp.float32)

        def zero_row(r, carry):
            def zero_col(j, carry2):
                zbuf[r, pl.ds(j * 16, 16)] = z
                return carry2
            return lax.fori_loop(0, D // 16, zero_col, carry)

        lax.fori_loop(0, RK, zero_row, 0)

        nch = (RCH - s + NS - 1) // NS

        def zero_piece(p, carry):
            r0 = pl.multiple_of((s + p * NS) * RK, 8)
            pltpu.async_copy(zbuf, acc.at[pl.ds(r0, RK)], sc_sem.at[0])
            return carry

        def zero_drain(p, carry):
            pltpu.make_async_copy(zbuf, acc.at[pl.ds(0, RK)],
                                  sc_sem.at[0]).wait()
            return carry

        lax.fori_loop(0, nch, zero_piece, 0)
        lax.fori_loop(0, nch, zero_drain, 0)
        plsc.subcore_barrier()

        def group(g, carry):
            for b in range(NB):
                j = g * NB + b

                @pl.when(j < C)
                def _():
                    wait_loads(b)
                    pltpu.async_copy(rows_v.at[b], acc.at[idx_v.at[b]],
                                     sc_sem.at[b], add=True)
            for b in range(NB):
                j = g * NB + b
                jn = (g + 1) * NB + b

                @pl.when(j < C)
                def _():
                    wait_scatter(b)

                @pl.when(jn < C)
                def _():
                    issue_loads(jn, b)
            return carry

        lax.fori_loop(0, NG, group, 0)

        if T:
            e0 = pl.multiple_of(cbase + C * K, 8)
            r0 = pl.multiple_of(rbase + C * K, 8)
            pltpu.sync_copy(col_hbm.at[pl.ds(e0, T)], idx_t)
            pltpu.sync_copy(ea_hbm.at[pl.ds(r0, T)], zbuf.at[pl.ds(0, T)])
            pltpu.sync_copy(zbuf.at[pl.ds(0, T)], acc.at[idx_t], add=True)

        plsc.subcore_barrier()

        def write_piece(p, carry):
            r0 = pl.multiple_of((s + p * NS) * RK, 8)
            pltpu.async_copy(acc.at[pl.ds(r0, RK)], out_hbm.at[c, pl.ds(r0, RK)],
                             sc_sem.at[0])
            return carry

        def write_drain(p, carry):
            pltpu.make_async_copy(acc.at[pl.ds(0, RK)],
                                  out_hbm.at[c, pl.ds(0, RK)],
                                  sc_sem.at[0]).wait()
            return carry

        lax.fori_loop(0, nch, write_piece, 0)
        lax.fori_loop(0, nch, write_drain, 0)

    return scatter_kernel


def _mlp(x, partial, W1, b1, W2, b2):
    N, Din = x.shape
    H = W1.shape[1]
    Dout = W2.shape[1]
    R = 400

    def body(x_ref, p_ref, w1_ref, b1_ref, w2_ref, b2_ref, o_ref):
        out1 = p_ref[0] + p_ref[1]
        hcat = jnp.concatenate([x_ref[...], out1], axis=1)
        h = jnp.dot(hcat, w1_ref[...], preferred_element_type=jnp.float32)
        h = jnp.maximum(h + b1_ref[...], 0.0)
        h = jnp.dot(h, w2_ref[...], preferred_element_type=jnp.float32)
        o_ref[...] = h + b2_ref[...] + x_ref[...]

    return pl.pallas_call(
        body,
        grid=(N // R,),
        in_specs=[
            pl.BlockSpec((R, Din), lambda i: (i, 0)),
            pl.BlockSpec((2, R, Din), lambda i: (0, i, 0)),
            pl.BlockSpec((2 * Din, H), lambda i: (0, 0)),
            pl.BlockSpec((1, H), lambda i: (0, 0)),
            pl.BlockSpec((H, Dout), lambda i: (0, 0)),
            pl.BlockSpec((1, Dout), lambda i: (0, 0)),
        ],
        out_specs=pl.BlockSpec((R, Dout), lambda i: (i, 0)),
        out_shape=jax.ShapeDtypeStruct((N, Dout), jnp.float32),
    )(x, partial, W1, b1.reshape(1, H), W2, b2.reshape(1, Dout))


def kernel(x, edge_index, edge_attr, u, batch, W1, b1, W2, b2):
    E, D = edge_attr.shape
    N = x.shape[0]
    info = plsc.get_sparse_core_info()
    NC, NS = info.num_cores, info.num_subcores
    scatter = _make_scatter(E, N, D, NC, NS, K=40, NB=8, RK=40)
    partial = scatter(edge_index.reshape(-1), edge_attr)
    return _mlp(x, partial, W1, b1, W2, b2)


# speedup vs baseline: 1.0683x; 1.0683x over previous
import functools

import jax
import jax.numpy as jnp
from jax import lax
from jax.experimental import pallas as pl
from jax.experimental.pallas import tpu as pltpu
from jax.experimental.pallas import tpu_sc as plsc


def _make_scatter(E, N, D, NC, NS, K, NB, RK=80):
    W = NC * NS
    EPW = E // W
    C = EPW // K
    T = EPW - C * K
    NG = (C + NB - 1) // NB
    RCH = N // RK
    mesh = plsc.VectorSubcoreMesh(core_axis_name="c", subcore_axis_name="s")

    @functools.partial(
        pl.kernel,
        out_type=jax.ShapeDtypeStruct((NC, N, D), jnp.float32),
        mesh=mesh,
        scratch_types=[
            pltpu.VMEM((NB, K), jnp.int32),
            pltpu.VMEM((NB, K, D), jnp.float32),
            pltpu.VMEM((16,), jnp.int32),
            pltpu.VMEM((RK, D), jnp.float32),
            pltpu.VMEM_SHARED((N, D), jnp.float32),
            pltpu.SemaphoreType.DMA((NB,)),
            pltpu.SemaphoreType.DMA((NB,)),
        ],
    )
    def scatter_kernel(col_hbm, ea_hbm, out_hbm, idx_v, rows_v, idx_t, zbuf,
                       acc, load_sem, sc_sem):
        c = lax.axis_index("c")
        s = lax.axis_index("s")
        wid = c * NS + s
        rbase = wid * EPW
        cbase = E + rbase

        def issue_loads(j, b):
            c0 = pl.multiple_of(cbase + j * K, 8)
            r0 = pl.multiple_of(rbase + j * K, 8)
            pltpu.async_copy(col_hbm.at[pl.ds(c0, K)], idx_v.at[b],
                             load_sem.at[b])
            pltpu.async_copy(ea_hbm.at[pl.ds(r0, K)], rows_v.at[b],
                             load_sem.at[b])

        def wait_loads(b):
            pltpu.make_async_copy(col_hbm.at[pl.ds(0, K)], idx_v.at[b],
                                  load_sem.at[b]).wait()
            pltpu.make_async_copy(ea_hbm.at[pl.ds(0, K)], rows_v.at[b],
                                  load_sem.at[b]).wait()

        def wait_scatter(b):
            pltpu.make_async_copy(rows_v.at[b], acc.at[idx_v.at[b]],
                                  sc_sem.at[b]).wait()

        for b in range(NB):
            issue_loads(b, b)

        z = jnp.zeros((16,), jnp.float32)

        def zero_row(r, carry):
            def zero_col(j, carry2):
                zbuf[r, pl.ds(j * 16, 16)] = z
                return carry2
            return lax.fori_loop(0, D // 16, zero_col, carry)

        lax.fori_loop(0, RK, zero_row, 0)

        nch = (RCH - s + NS - 1) // NS

        def zero_piece(p, carry):
            r0 = pl.multiple_of((s + p * NS) * RK, 8)
            pltpu.async_copy(zbuf, acc.at[pl.ds(r0, RK)], sc_sem.at[0])
            return carry

        def zero_drain(p, carry):
            pltpu.make_async_copy(zbuf, acc.at[pl.ds(0, RK)],
                                  sc_sem.at[0]).wait()
            return carry

        lax.fori_loop(0, nch, zero_piece, 0)
        lax.fori_loop(0, nch, zero_drain, 0)
        plsc.subcore_barrier()

        def group(g, carry):
            for b in range(NB):
                j = g * NB + b

                @pl.when(j < C)
                def _():
                    wait_loads(b)
                    pltpu.async_copy(rows_v.at[b], acc.at[idx_v.at[b]],
                                     sc_sem.at[b], add=True)
            for b in range(NB):
                j = g * NB + b
                jn = (g + 1) * NB + b

                @pl.when(j < C)
                def _():
                    wait_scatter(b)

                @pl.when(jn < C)
                def _():
                    issue_loads(jn, b)
            return carry

        lax.fori_loop(0, NG, group, 0)

        if T:
            e0 = pl.multiple_of(cbase + C * K, 8)
            r0 = pl.multiple_of(rbase + C * K, 8)
            pltpu.sync_copy(col_hbm.at[pl.ds(e0, T)], idx_t)
            pltpu.sync_copy(ea_hbm.at[pl.ds(r0, T)], zbuf.at[pl.ds(0, T)])
            pltpu.sync_copy(zbuf.at[pl.ds(0, T)], acc.at[idx_t], add=True)

        plsc.subcore_barrier()

        def write_piece(p, carry):
            r0 = pl.multiple_of((s + p * NS) * RK, 8)
            pltpu.async_copy(acc.at[pl.ds(r0, RK)], out_hbm.at[c, pl.ds(r0, RK)],
                             sc_sem.at[0])
            return carry

        def write_drain(p, carry):
            pltpu.make_async_copy(acc.at[pl.ds(0, RK)],
                                  out_hbm.at[c, pl.ds(0, RK)],
                                  sc_sem.at[0]).wait()
            return carry

        lax.fori_loop(0, nch, write_piece, 0)
        lax.fori_loop(0, nch, write_drain, 0)

    return scatter_kernel


def _mlp(x, partial, W1, b1, W2, b2):
    N, Din = x.shape
    H = W1.shape[1]
    Dout = W2.shape[1]
    R = 1000

    def body(x_ref, p_ref, w1_ref, b1_ref, w2_ref, b2_ref, o_ref):
        out1 = p_ref[0] + p_ref[1]
        hcat = jnp.concatenate([x_ref[...], out1], axis=1)
        h = jnp.dot(hcat, w1_ref[...], preferred_element_type=jnp.float32)
        h = jnp.maximum(h + b1_ref[...], 0.0)
        h = jnp.dot(h, w2_ref[...], preferred_element_type=jnp.float32)
        o_ref[...] = h + b2_ref[...] + x_ref[...]

    return pl.pallas_call(
        body,
        grid=(N // R,),
        in_specs=[
            pl.BlockSpec((R, Din), lambda i: (i, 0)),
            pl.BlockSpec((2, R, Din), lambda i: (0, i, 0)),
            pl.BlockSpec((2 * Din, H), lambda i: (0, 0)),
            pl.BlockSpec((1, H), lambda i: (0, 0)),
            pl.BlockSpec((H, Dout), lambda i: (0, 0)),
            pl.BlockSpec((1, Dout), lambda i: (0, 0)),
        ],
        out_specs=pl.BlockSpec((R, Dout), lambda i: (i, 0)),
        out_shape=jax.ShapeDtypeStruct((N, Dout), jnp.float32),
    )(x, partial, W1, b1.reshape(1, H), W2, b2.reshape(1, Dout))


def kernel(x, edge_index, edge_attr, u, batch, W1, b1, W2, b2):
    E, D = edge_attr.shape
    N = x.shape[0]
    info = plsc.get_sparse_core_info()
    NC, NS = info.num_cores, info.num_subcores
    scatter = _make_scatter(E, N, D, NC, NS, K=40, NB=8, RK=40)
    partial = scatter(edge_index.reshape(-1), edge_attr)
    return _mlp(x, partial, W1, b1, W2, b2)
